# baseline (speedup 1.0000x reference)
import functools

import jax
import jax.numpy as jnp
from jax import lax
from jax.experimental import pallas as pl
from jax.experimental.pallas import tpu as pltpu
from jax.experimental.pallas import tpu_sc as plsc

NC = 2
NS = 16
NW = NC * NS

CHUNK = 128
NBUF = 7
GPRE = 4


def _make_gather(total_rows: int, dim: int):
  assert total_rows % NW == 0
  rows_per_w = total_rows // NW
  assert rows_per_w % CHUNK == 0
  nchunk = rows_per_w // CHUNK

  mesh = plsc.VectorSubcoreMesh(core_axis_name="c", subcore_axis_name="s")

  @functools.partial(
      pl.kernel,
      out_type=jax.ShapeDtypeStruct((total_rows, dim), jnp.float32),
      mesh=mesh,
      scratch_types=[
          pltpu.VMEM((nchunk, CHUNK), jnp.int32),
          [pltpu.VMEM((CHUNK, dim), jnp.float32) for _ in range(NBUF)],
          [pltpu.SemaphoreType.DMA for _ in range(NBUF)],
          [pltpu.SemaphoreType.DMA for _ in range(NBUF)],
      ],
  )
  def gather_kernel(table_hbm, idx_hbm, out_hbm, idx_v, bufs, in_sems, out_sems):
    wid = lax.axis_index("s") * NC + lax.axis_index("c")
    row_base = wid * rows_per_w

    pltpu.sync_copy(idx_hbm.at[wid], idx_v)

    def start_gather(chunk):
      b = chunk % NBUF
      pltpu.async_copy(table_hbm.at[idx_v.at[chunk]], bufs[b], in_sems[b])

    def wait_sem(sems, b):
      pltpu.make_async_copy(table_hbm.at[pl.ds(0, CHUNK)], bufs[b], sems[b]).wait()

    def start_out(chunk):
      b = chunk % NBUF
      pltpu.async_copy(bufs[b], out_hbm.at[pl.ds(row_base + chunk * CHUNK, CHUNK)],
                       out_sems[b])

    for c in range(GPRE):
      start_gather(c)
    for i in range(nchunk):
      nxt = i + GPRE
      if nxt < nchunk:
        if nxt >= NBUF:
          wait_sem(out_sems, nxt % NBUF)
        start_gather(nxt)
      wait_sem(in_sems, i % NBUF)
      start_out(i)
    for i in range(max(0, nchunk - NBUF), nchunk):
      wait_sem(out_sems, i % NBUF)

  return gather_kernel


@jax.jit
def kernel(x, table):
  batch, seq = x.shape
  dim = table.shape[1]
  total = batch * seq
  idx = x.astype(jnp.int32).reshape(NW, total // NW // CHUNK, CHUNK)
  out = _make_gather(total, dim)(table, idx)
  return out.reshape(batch, seq, dim)


# speedup vs baseline: 1.7852x; 1.7852x over previous
import functools

import jax
import jax.numpy as jnp
from jax import lax
from jax.experimental import pallas as pl
from jax.experimental.pallas import tpu as pltpu
from jax.experimental.pallas import tpu_sc as plsc

NC = 2
NS = 16
NW = NC * NS

NBUF = 8
GPRE = 5


def _make_gather(batch: int, seq: int, dim: int):
  assert batch % NW == 0
  bpw = batch // NW

  mesh = plsc.VectorSubcoreMesh(core_axis_name="c", subcore_axis_name="s")

  @functools.partial(
      pl.kernel,
      out_type=jax.ShapeDtypeStruct((batch, seq, dim), jnp.float32),
      mesh=mesh,
      scratch_types=[
          pltpu.VMEM((bpw, seq), jnp.int32),
          [pltpu.VMEM((seq, dim), jnp.float32) for _ in range(NBUF)],
          [pltpu.SemaphoreType.DMA for _ in range(NBUF)],
          [pltpu.SemaphoreType.DMA for _ in range(NBUF)],
      ],
  )
  def gather_kernel(table_hbm, idx_hbm, out_hbm, idx_v, bufs, in_sems, out_sems):
    wid = lax.axis_index("s") * NC + lax.axis_index("c")
    batch_base = wid * bpw

    pltpu.sync_copy(idx_hbm.at[pl.ds(batch_base, bpw)], idx_v)

    def start_gather(j, b):
      pltpu.async_copy(table_hbm.at[idx_v.at[j]], bufs[b], in_sems[b])

    def wait_sem(sems, b):
      pltpu.make_async_copy(out_hbm.at[0], bufs[b], sems[b]).wait()

    def start_out(j, b):
      pltpu.async_copy(bufs[b], out_hbm.at[batch_base + j], out_sems[b])

    for j in range(GPRE):
      start_gather(j, j % NBUF)

    @pl.loop(0, bpw, step=NBUF)
    def _(g):
      for b in range(NBUF):
        j = g + b

        bn = (b + GPRE) % NBUF

        @pl.when(j + GPRE < bpw)
        def _():

          @pl.when(j + GPRE >= NBUF)
          def _():
            wait_sem(out_sems, bn)

          start_gather(j + GPRE, bn)

        wait_sem(in_sems, b)
        start_out(j, b)

    for b in range(NBUF):
      wait_sem(out_sems, b)

  return gather_kernel


@jax.jit
def kernel(x, table):
  batch, seq = x.shape
  dim = table.shape[1]
  return _make_gather(batch, seq, dim)(table, x.astype(jnp.int32))


# speedup vs baseline: 1.8005x; 1.0085x over previous
import functools

import jax
import jax.numpy as jnp
from jax import lax
from jax.experimental import pallas as pl
from jax.experimental.pallas import tpu as pltpu
from jax.experimental.pallas import tpu_sc as plsc

NC = 2
NS = 16
NW = NC * NS

NB = 2
NBUF = 8
GPRE = 5


def _make_gather(batch: int, seq: int, dim: int):
  assert batch % (NW * NB) == 0
  bpw = batch // NW
  ng = bpw // NB
  assert ng % NBUF == 0
  assert NB * seq <= 128

  mesh = plsc.VectorSubcoreMesh(core_axis_name="c", subcore_axis_name="s")

  @functools.partial(
      pl.kernel,
      out_type=jax.ShapeDtypeStruct((batch, seq, dim), jnp.float32),
      mesh=mesh,
      scratch_types=[
          pltpu.VMEM((ng, NB * seq), jnp.int32),
          [pltpu.VMEM((NB * seq, dim), jnp.float32) for _ in range(NBUF)],
          [pltpu.SemaphoreType.DMA for _ in range(NBUF)],
          [pltpu.SemaphoreType.DMA for _ in range(NBUF)],
      ],
  )
  def gather_kernel(table_hbm, idx_hbm, out_hbm, idx_v, bufs, in_sems, out_sems):
    wid = lax.axis_index("s") * NC + lax.axis_index("c")
    batch_base = wid * bpw

    pltpu.sync_copy(idx_hbm.at[wid], idx_v)

    def start_gather(j, b):
      pltpu.async_copy(table_hbm.at[idx_v.at[j]], bufs[b], in_sems[b])

    def wait_sem(sems, b):
      pltpu.make_async_copy(out_hbm.at[pl.ds(0, NB)],
                            bufs[b].reshape(NB, seq, dim), sems[b]).wait()

    def start_out(j, b):
      pltpu.async_copy(bufs[b].reshape(NB, seq, dim),
                       out_hbm.at[pl.ds(batch_base + j * NB, NB)],
                       out_sems[b])

    for j in range(GPRE):
      start_gather(j, j % NBUF)

    @pl.loop(0, ng, step=NBUF)
    def _(g):
      for b in range(NBUF):
        j = g + b
        bn = (b + GPRE) % NBUF

        @pl.when(j + GPRE < ng)
        def _():
          @pl.when(j + GPRE >= NBUF)
          def _():
            wait_sem(out_sems, bn)

          start_gather(j + GPRE, bn)

        wait_sem(in_sems, b)
        start_out(j, b)

    for b in range(NBUF):
      wait_sem(out_sems, b)

  return gather_kernel


@jax.jit
def kernel(x, table):
  batch, seq = x.shape
  dim = table.shape[1]
  ng = batch // NW // NB
  xr = x.astype(jnp.int32).reshape(NW, ng, NB * seq)
  return _make_gather(batch, seq, dim)(table, xr)


# speedup vs baseline: 3.2021x; 1.7785x over previous
import functools

import jax
import jax.numpy as jnp
from jax import lax
from jax.experimental import pallas as pl
from jax.experimental.pallas import tpu as pltpu
from jax.experimental.pallas import tpu_sc as plsc

NC = 2
NS = 16
NW = NC * NS

NBUF = 5
GPRE = 3


def _make_gather(batch: int, seq: int, dim: int):
  assert batch % NW == 0
  bpw = batch // NW
  assert bpw <= 128
  assert seq % NBUF == 0

  mesh = plsc.VectorSubcoreMesh(core_axis_name="c", subcore_axis_name="s")

  @functools.partial(
      pl.kernel,
      out_type=jax.ShapeDtypeStruct((seq, batch, dim), jnp.float32),
      mesh=mesh,
      scratch_types=[
          pltpu.VMEM((seq, bpw), jnp.int32),
          [pltpu.VMEM((bpw, dim), jnp.float32) for _ in range(NBUF)],
          [pltpu.SemaphoreType.DMA for _ in range(NBUF)],
          [pltpu.SemaphoreType.DMA for _ in range(NBUF)],
      ],
  )
  def gather_kernel(table_hbm, idx_hbm, out_hbm, idx_v, bufs, in_sems, out_sems):
    wid = lax.axis_index("s") * NC + lax.axis_index("c")
    batch_base = wid * bpw

    pltpu.sync_copy(idx_hbm.at[:, wid], idx_v)

    def start_gather(l, b):
      pltpu.async_copy(table_hbm.at[idx_v.at[l]], bufs[b], in_sems[b])

    def wait_sem(sems, b):
      pltpu.make_async_copy(table_hbm.at[pl.ds(0, bpw)], bufs[b], sems[b]).wait()

    def start_out(l, b):
      pltpu.async_copy(bufs[b], out_hbm.at[l, pl.ds(batch_base, bpw)],
                       out_sems[b])

    for l in range(GPRE):
      start_gather(l, l % NBUF)

    @pl.loop(0, seq, step=NBUF)
    def _(g):
      for b in range(NBUF):
        l = g + b
        bn = (b + GPRE) % NBUF

        @pl.when(l + GPRE < seq)
        def _():
          @pl.when(l + GPRE >= NBUF)
          def _():
            wait_sem(out_sems, bn)

          start_gather(l + GPRE, bn)

        wait_sem(in_sems, b)
        start_out(l, b)

    for b in range(NBUF):
      wait_sem(out_sems, b)

  return gather_kernel


@jax.jit
def kernel(x, table):
  batch, seq = x.shape
  dim = table.shape[1]
  xt = x.astype(jnp.int32).T.reshape(seq, NW, batch // NW)
  out = _make_gather(batch, seq, dim)(table, xt)
  return out.transpose(1, 0, 2)
